# baseline (speedup 1.0000x reference)
import functools

import jax
import jax.numpy as jnp
from jax import lax
from jax.experimental import pallas as pl
from jax.experimental.pallas import tpu as pltpu
from jax.experimental.pallas import tpu_sc as plsc

N = 10000
NP = 10240
D = 64
E = 320000
NT = 16
NCH = 160
CK = 128
NG8 = NCH // 8
RPT = NP // NT
NRC = RPT // CK
PAD_NODE = N + 200
NG = 64

_mesh = plsc.VectorSubcoreMesh(core_axis_name="c", subcore_axis_name="s",
                               num_cores=1)


def _rsqrt16(d):
    s = 0.5 * (d + 1.0)
    for _ in range(12):
        s = 0.5 * (s + d / s)
    return 1.0 / s


def _bcast_lane15(v):
    return jnp.full((16,), v[15])


_LANE15 = None


@functools.partial(
    pl.kernel,
    mesh=_mesh,
    out_type=[jax.ShapeDtypeStruct((NP, D), jnp.float32),
              jax.ShapeDtypeStruct((NP, D), jnp.float32)],
    compiler_params=pltpu.CompilerParams(use_tc_tiling_on_sc=False),
    scratch_types=[
        pltpu.VMEM((NCH, CK), jnp.int32),
        pltpu.VMEM((NCH, CK), jnp.int32),
        pltpu.VMEM((CK, D), jnp.float32),
        pltpu.VMEM((CK, D), jnp.float32),
        pltpu.VMEM((D * D,), jnp.float32),
        pltpu.VMEM((D,), jnp.float32),
        pltpu.VMEM_SHARED((NP, D), jnp.float32),
        pltpu.SemaphoreType.DMA,
        pltpu.SemaphoreType.DMA,
    ],
)
def _gcn_sc(t1, srcm, dstm, w2, w3, b1, b2, b3, out, s_hbm,
            src_v, dst_v, buf0, buf1, wv, bv, q_sh, sem0, sem1):
    sid = lax.axis_index("s")
    row0 = sid * RPT
    lane15 = lax.iota(jnp.int32, 16) == 15

    def fill(buf, val):
        v = jnp.full((16,), val, jnp.float32)

        def body(r, carry):
            for w in range(4):
                buf[r, pl.ds(16 * w, 16)] = v
            return carry

        lax.fori_loop(0, CK, body, 0)

    def zero_q_chunk(c):
        pltpu.sync_copy(buf0, q_sh.at[pl.ds(row0 + c * CK, CK)])

    pltpu.sync_copy(srcm.at[sid], src_v)
    pltpu.sync_copy(dstm.at[sid], dst_v)
    fill(buf0, 0.0)

    def p0_body(c, carry):
        zero_q_chunk(c)
        return carry

    lax.fori_loop(0, NRC, p0_body, 0)
    fill(buf0, 1.0)
    plsc.subcore_barrier()

    def hist_body(j, carry):
        pltpu.sync_copy(buf0, q_sh.at[dst_v.at[j]], add=True)
        return carry

    lax.fori_loop(0, NCH, hist_body, 0)
    plsc.subcore_barrier()

    def stage_body(c, carry):
        r0 = row0 + c * CK
        pltpu.sync_copy(q_sh.at[pl.ds(r0, CK)], buf0)
        pltpu.sync_copy(t1.at[pl.ds(r0, CK)], buf1)

        def row_body(r, c2):
            y = _rsqrt16(buf0[r, pl.ds(0, 16)] + 1.0)
            for w in range(4):
                sl = pl.ds(16 * w, 16)
                v = y * buf1[r, sl]
                if w == 3:
                    v = jnp.where(lane15, y, v)
                buf1[r, sl] = v
            return c2

        lax.fori_loop(0, CK, row_body, 0)
        pltpu.sync_copy(buf1, s_hbm.at[pl.ds(r0, CK)])
        return carry

    lax.fori_loop(0, NRC, stage_body, 0)
    fill(buf0, 0.0)

    def rezero_body(c, carry):
        zero_q_chunk(c)
        return carry

    lax.fori_loop(0, NRC, rezero_body, 0)
    plsc.subcore_barrier()

    def prop():
        pltpu.async_copy(s_hbm.at[src_v.at[0]], buf0, sem0)

        def pair(i, carry):
            j = 2 * i
            pltpu.async_copy(s_hbm.at[src_v.at[j + 1]], buf1, sem1)
            pltpu.make_async_copy(s_hbm.at[src_v.at[j]], buf0, sem0).wait()
            pltpu.sync_copy(buf0, q_sh.at[dst_v.at[j]], add=True)

            @pl.when(j + 2 < NCH)
            def _():
                pltpu.async_copy(s_hbm.at[src_v.at[j + 2]], buf0, sem0)

            pltpu.make_async_copy(s_hbm.at[src_v.at[j + 1]], buf1, sem1).wait()
            pltpu.sync_copy(buf1, q_sh.at[dst_v.at[j + 1]], add=True)
            return carry

        lax.fori_loop(0, NCH // 2, pair, 0)
        plsc.subcore_barrier()

    def load_m(c):
        r0 = row0 + c * CK
        pltpu.sync_copy(q_sh.at[pl.ds(r0, CK)], buf0)
        pltpu.sync_copy(s_hbm.at[pl.ds(r0, CK)], buf1)

        def row_body(r, c2):
            dv = _bcast_lane15(buf1[r, pl.ds(48, 16)])
            for w in range(4):
                sl = pl.ds(16 * w, 16)
                buf0[r, sl] = dv * (buf0[r, sl] + buf1[r, sl])
            return c2

        lax.fori_loop(0, CK, row_body, 0)

    def finish_chunk(c):
        r0 = row0 + c * CK
        pltpu.sync_copy(buf1, s_hbm.at[pl.ds(r0, CK)])
        fill(buf0, 0.0)
        zero_q_chunk(c)

    def trans_elem():
        def c_body(c, carry):
            load_m(c)

            def row_body(r, c2):
                dv = _bcast_lane15(buf1[r, pl.ds(48, 16)])
                for w in range(4):
                    sl = pl.ds(16 * w, 16)
                    h = jnp.maximum(buf0[r, sl] + bv[pl.ds(16 * w, 16)], 0.0)
                    v = dv * h
                    if w == 3:
                        v = jnp.where(lane15, dv, v)
                    buf1[r, sl] = v
                return c2

            lax.fori_loop(0, CK, row_body, 0)
            finish_chunk(c)
            return carry

        lax.fori_loop(0, NRC, c_body, 0)
        plsc.subcore_barrier()

    def trans_matmul():
        def c_body(c, carry):
            load_m(c)

            def g_body(g, carry2):
                rg = 8 * g

                def k16_body(k16, accs):
                    accs = list(accs)
                    mvs = [buf0[rg + rr, pl.ds(16 * k16, 16)]
                           for rr in range(8)]
                    for k in range(16):
                        col = 16 * k16 + k
                        wr = [wv[pl.ds(col * D + 16 * w, 16)]
                              for w in range(4)]
                        for rr in range(8):
                            bb = jnp.full((16,), mvs[rr][k])
                            for w in range(4):
                                accs[rr * 4 + w] = accs[rr * 4 + w] + bb * wr[w]
                    return tuple(accs)

                zero = jnp.zeros((16,), jnp.float32)
                accs = lax.fori_loop(0, 4, k16_body, (zero,) * 32)
                for rr in range(8):
                    dv = _bcast_lane15(buf1[rg + rr, pl.ds(48, 16)])
                    for w in range(4):
                        sl = pl.ds(16 * w, 16)
                        h = jnp.maximum(
                            accs[rr * 4 + w] + bv[pl.ds(16 * w, 16)], 0.0)
                        v = dv * h
                        if w == 3:
                            v = jnp.where(lane15, dv, v)
                        buf1[rg + rr, sl] = v
                return carry2

            lax.fori_loop(0, CK // 8, g_body, 0)
            finish_chunk(c)
            return carry

        lax.fori_loop(0, NRC, c_body, 0)
        plsc.subcore_barrier()

    prop()
    pltpu.sync_copy(b1, bv)
    trans_elem()
    prop()
    pltpu.sync_copy(w2, wv)
    pltpu.sync_copy(b2, bv)
    trans_matmul()
    prop()
    pltpu.sync_copy(w3, wv)
    pltpu.sync_copy(b3, bv)
    trans_matmul()
    prop()

    def out_body(c, carry):
        r0 = row0 + c * CK
        load_m(c)
        pltpu.sync_copy(buf0, out.at[pl.ds(r0, CK)])
        return carry

    lax.fori_loop(0, NRC, out_body, 0)


def _prep_body(x_ref, w1_ref, s1_ref):
    s1_ref[...] = jnp.dot(x_ref[...], w1_ref[...],
                          preferred_element_type=jnp.float32)


def _final_body(m4_ref, w4_ref, b4_ref, batch_ref,
                fw1_ref, fb1_ref, fw2_ref, fb2_ref, out_ref):
    h4 = jnp.maximum(
        jnp.dot(m4_ref[...], w4_ref[...], preferred_element_type=jnp.float32)
        + b4_ref[...], 0.0)
    g = lax.broadcasted_iota(jnp.int32, (NP, NG), 1)
    oh = (batch_ref[...] == g).astype(jnp.float32)
    sums = lax.dot_general(oh, h4, (((0,), (0,)), ((), ())),
                           preferred_element_type=jnp.float32)
    counts = jnp.sum(oh, axis=0)[:, None]
    pooled = sums / jnp.maximum(counts, 1.0)
    pooled = pooled.astype(jnp.bfloat16).astype(jnp.float32)
    z = jnp.dot(pooled, fw1_ref[...], preferred_element_type=jnp.float32) + fb1_ref[...]
    z = z.astype(jnp.bfloat16).astype(jnp.float32)
    out_ref[...] = jnp.dot(z, fw2_ref[...], preferred_element_type=jnp.float32) + fb2_ref[...]


def kernel(x, edge_index, batch, W1, b1, W2, b2, W3, b3, W4, b4,
           fcW1, fcb1, fcW2, fcb2):
    f = W1.shape[1]
    def rr(a):
        return a.astype(jnp.bfloat16).astype(jnp.float32)

    xp = rr(jnp.pad(x, ((0, NP - N), (0, 0))))
    padlen = NT * NCH * CK - E
    pad = jnp.full((padlen,), PAD_NODE, jnp.int32)
    srcm = jnp.concatenate([edge_index[0], pad]).reshape(NT, NCH, CK)
    dstm = jnp.concatenate([edge_index[1], pad]).reshape(NT, NCH, CK)
    batch_p = jnp.pad(batch, (0, NP - N), constant_values=NG).reshape(NP, 1)
    W1p = rr(jnp.pad(W1, ((0, 0), (0, D - f))))
    W2f = rr(jnp.pad(W2, ((0, D - f), (0, D - f)))).reshape(-1)
    W3f = rr(jnp.pad(W3, ((0, D - f), (0, D - f)))).reshape(-1)
    W4p = rr(jnp.pad(W4, ((0, D - f), (0, 0))))
    b1p = jnp.pad(b1, (0, D - f))
    b2p = jnp.pad(b2, (0, D - f))
    b3p = jnp.pad(b3, (0, D - f))
    b4r = b4.reshape(1, -1)
    fcb1r = fcb1.reshape(1, -1)
    fcb2r = fcb2.reshape(1, -1)

    t1 = pl.pallas_call(
        _prep_body,
        out_shape=jax.ShapeDtypeStruct((NP, D), jnp.float32))(xp, W1p)
    m4, _ = _gcn_sc(t1, srcm, dstm, W2f, W3f, b1p, b2p, b3p)
    z = pl.pallas_call(
        _final_body,
        out_shape=jax.ShapeDtypeStruct((NG, 10), jnp.float32))(
            m4, W4p, b4r, batch_p, rr(fcW1), fcb1r, rr(fcW2), fcb2r)
    return z


# speedup vs baseline: 1.2580x; 1.2580x over previous
import functools

import jax
import jax.numpy as jnp
from jax import lax
from jax.experimental import pallas as pl
from jax.experimental.pallas import tpu as pltpu
from jax.experimental.pallas import tpu_sc as plsc

N = 10000
NP = 10240
D = 64
E = 320000
NT = 16
NCH = 160
CK = 128
NG8 = NCH // 8
RPT = NP // NT
NRC = RPT // CK
PAD_NODE = N + 200
NG = 64

_mesh = plsc.VectorSubcoreMesh(core_axis_name="c", subcore_axis_name="s",
                               num_cores=1)


def _rsqrt16(d):
    s = 0.5 * (d + 1.0)
    for _ in range(12):
        s = 0.5 * (s + d / s)
    return 1.0 / s


def _bcast_lane15(v):
    return jnp.full((16,), v[15])


_LANE15 = None


@functools.partial(
    pl.kernel,
    mesh=_mesh,
    out_type=[jax.ShapeDtypeStruct((NP, D), jnp.float32),
              jax.ShapeDtypeStruct((NP, D), jnp.float32)],
    compiler_params=pltpu.CompilerParams(use_tc_tiling_on_sc=False),
    scratch_types=[
        pltpu.VMEM((NCH, CK), jnp.int32),
        pltpu.VMEM((NCH, CK), jnp.int32),
        pltpu.VMEM((CK, D), jnp.float32),
        pltpu.VMEM((CK, D), jnp.float32),
        pltpu.VMEM((D * D,), jnp.float32),
        pltpu.VMEM((D,), jnp.float32),
        pltpu.VMEM_SHARED((NP, D), jnp.float32),
        pltpu.SemaphoreType.DMA,
        pltpu.SemaphoreType.DMA,
    ],
)
def _gcn_sc(t1, srcm, dstm, w2, w3, b1, b2, b3, out, s_hbm,
            src_v, dst_v, buf0, buf1, wv, bv, q_sh, sem0, sem1):
    sid = lax.axis_index("s")
    row0 = sid * RPT
    lane15 = lax.iota(jnp.int32, 16) == 15

    def fill(buf, val):
        v = jnp.full((16,), val, jnp.float32)

        def body(r, carry):
            for w in range(4):
                buf[r, pl.ds(16 * w, 16)] = v
            return carry

        lax.fori_loop(0, CK, body, 0)

    def zero_q_chunk(c):
        pltpu.sync_copy(buf0, q_sh.at[pl.ds(row0 + c * CK, CK)])

    pltpu.sync_copy(srcm.at[sid], src_v)
    pltpu.sync_copy(dstm.at[sid], dst_v)
    fill(buf0, 0.0)

    def p0_body(c, carry):
        zero_q_chunk(c)
        return carry

    lax.fori_loop(0, NRC, p0_body, 0)
    fill(buf0, 1.0)
    plsc.subcore_barrier()

    def hist_body(j, carry):
        pltpu.sync_copy(buf0, q_sh.at[dst_v.at[j]], add=True)
        return carry

    lax.fori_loop(0, NCH, hist_body, 0)
    plsc.subcore_barrier()

    def stage_body(c, carry):
        r0 = row0 + c * CK
        pltpu.sync_copy(q_sh.at[pl.ds(r0, CK)], buf0)
        pltpu.sync_copy(t1.at[pl.ds(r0, CK)], buf1)

        def row_body(r, c2):
            y = _rsqrt16(buf0[r, pl.ds(0, 16)] + 1.0)
            for w in range(4):
                sl = pl.ds(16 * w, 16)
                v = y * buf1[r, sl]
                if w == 3:
                    v = jnp.where(lane15, y, v)
                buf1[r, sl] = v
            return c2

        lax.fori_loop(0, CK, row_body, 0)
        pltpu.sync_copy(buf1, s_hbm.at[pl.ds(r0, CK)])
        return carry

    lax.fori_loop(0, NRC, stage_body, 0)
    fill(buf0, 0.0)

    def rezero_body(c, carry):
        zero_q_chunk(c)
        return carry

    lax.fori_loop(0, NRC, rezero_body, 0)
    plsc.subcore_barrier()

    def prop():
        pltpu.async_copy(s_hbm.at[src_v.at[0]], buf0, sem0)

        def pair(i, carry):
            j = 2 * i
            pltpu.async_copy(s_hbm.at[src_v.at[j + 1]], buf1, sem1)
            pltpu.make_async_copy(s_hbm.at[src_v.at[j]], buf0, sem0).wait()
            pltpu.sync_copy(buf0, q_sh.at[dst_v.at[j]], add=True)

            @pl.when(j + 2 < NCH)
            def _():
                pltpu.async_copy(s_hbm.at[src_v.at[j + 2]], buf0, sem0)

            pltpu.make_async_copy(s_hbm.at[src_v.at[j + 1]], buf1, sem1).wait()
            pltpu.sync_copy(buf1, q_sh.at[dst_v.at[j + 1]], add=True)
            return carry

        lax.fori_loop(0, NCH // 2, pair, 0)
        plsc.subcore_barrier()

    def load_m(c):
        r0 = row0 + c * CK
        pltpu.sync_copy(q_sh.at[pl.ds(r0, CK)], buf0)
        pltpu.sync_copy(s_hbm.at[pl.ds(r0, CK)], buf1)

        def row_body(r, c2):
            dv = _bcast_lane15(buf1[r, pl.ds(48, 16)])
            for w in range(4):
                sl = pl.ds(16 * w, 16)
                buf0[r, sl] = dv * (buf0[r, sl] + buf1[r, sl])
            return c2

        lax.fori_loop(0, CK, row_body, 0)

    def finish_chunk(c):
        r0 = row0 + c * CK
        pltpu.sync_copy(buf1, s_hbm.at[pl.ds(r0, CK)])
        fill(buf0, 0.0)
        zero_q_chunk(c)

    def trans_elem():
        def c_body(c, carry):
            load_m(c)

            def row_body(r, c2):
                dv = _bcast_lane15(buf1[r, pl.ds(48, 16)])
                for w in range(4):
                    sl = pl.ds(16 * w, 16)
                    h = jnp.maximum(buf0[r, sl] + bv[pl.ds(16 * w, 16)], 0.0)
                    v = dv * h
                    if w == 3:
                        v = jnp.where(lane15, dv, v)
                    buf1[r, sl] = v
                return c2

            lax.fori_loop(0, CK, row_body, 0)
            finish_chunk(c)
            return carry

        lax.fori_loop(0, NRC, c_body, 0)
        plsc.subcore_barrier()

    def trans_matmul():
        def c_body(c, carry):
            load_m(c)

            def g_body(g, carry2):
                rg = 8 * g

                def k16_body(k16, accs):
                    accs = list(accs)
                    mvs = [buf0[rg + rr, pl.ds(16 * k16, 16)]
                           for rr in range(8)]
                    for k in range(16):
                        col = 16 * k16 + k
                        wr = [wv[pl.ds(col * D + 16 * w, 16)]
                              for w in range(4)]
                        for rr in range(8):
                            bb = jnp.full((16,), mvs[rr][k])
                            for w in range(4):
                                accs[rr * 4 + w] = accs[rr * 4 + w] + bb * wr[w]
                    return tuple(accs)

                zero = jnp.zeros((16,), jnp.float32)
                accs = lax.fori_loop(0, 4, k16_body, (zero,) * 32)
                for rr in range(8):
                    dv = _bcast_lane15(buf1[rg + rr, pl.ds(48, 16)])
                    for w in range(4):
                        sl = pl.ds(16 * w, 16)
                        h = jnp.maximum(
                            accs[rr * 4 + w] + bv[pl.ds(16 * w, 16)], 0.0)
                        v = dv * h
                        if w == 3:
                            v = jnp.where(lane15, dv, v)
                        buf1[rg + rr, sl] = v
                return carry2

            lax.fori_loop(0, CK // 8, g_body, 0)
            finish_chunk(c)
            return carry

        lax.fori_loop(0, NRC, c_body, 0)
        plsc.subcore_barrier()

    prop()
    pltpu.sync_copy(b1, bv)
    trans_elem()
    prop()
    pltpu.sync_copy(w2, wv)
    pltpu.sync_copy(b2, bv)
    trans_matmul()
    prop()
    pltpu.sync_copy(w3, wv)
    pltpu.sync_copy(b3, bv)
    trans_matmul()
    prop()

    def out_body(c, carry):
        r0 = row0 + c * CK
        load_m(c)
        pltpu.sync_copy(buf0, out.at[pl.ds(r0, CK)])
        return carry

    lax.fori_loop(0, NRC, out_body, 0)


def _prep_body(x_ref, w1_ref, s1_ref):
    s1_ref[...] = jnp.dot(x_ref[...], w1_ref[...],
                          preferred_element_type=jnp.float32)


def _final_body(m4_ref, w4_ref, b4_ref, batch_ref,
                fw1_ref, fb1_ref, fw2_ref, fb2_ref, out_ref):
    h4 = jnp.maximum(
        jnp.dot(m4_ref[...], w4_ref[...], preferred_element_type=jnp.float32)
        + b4_ref[...], 0.0)
    g = lax.broadcasted_iota(jnp.int32, (NP, NG), 1)
    oh = (batch_ref[...] == g).astype(jnp.float32)
    sums = lax.dot_general(oh, h4, (((0,), (0,)), ((), ())),
                           preferred_element_type=jnp.float32)
    counts = jnp.sum(oh, axis=0)[:, None]
    pooled = sums / jnp.maximum(counts, 1.0)
    z = jnp.dot(pooled, fw1_ref[...], preferred_element_type=jnp.float32) + fb1_ref[...]
    out_ref[...] = jnp.dot(z, fw2_ref[...], preferred_element_type=jnp.float32) + fb2_ref[...]


def kernel(x, edge_index, batch, W1, b1, W2, b2, W3, b3, W4, b4,
           fcW1, fcb1, fcW2, fcb2):
    f = W1.shape[1]
    def rr(a):
        return a.astype(jnp.bfloat16).astype(jnp.float32)

    xp = jnp.pad(x, ((0, NP - N), (0, 0)))
    padlen = NT * NCH * CK - E
    pad = jnp.full((padlen,), PAD_NODE, jnp.int32)
    srcm = jnp.concatenate([edge_index[0], pad]).reshape(NT, NCH, CK)
    dstm = jnp.concatenate([edge_index[1], pad]).reshape(NT, NCH, CK)
    batch_p = jnp.pad(batch, (0, NP - N), constant_values=NG).reshape(NP, 1)
    W1p = jnp.pad(W1, ((0, 0), (0, D - f)))
    W2f = rr(jnp.pad(W2, ((0, D - f), (0, D - f)))).reshape(-1)
    W3f = rr(jnp.pad(W3, ((0, D - f), (0, D - f)))).reshape(-1)
    W4p = jnp.pad(W4, ((0, D - f), (0, 0)))
    b1p = jnp.pad(b1, (0, D - f))
    b2p = jnp.pad(b2, (0, D - f))
    b3p = jnp.pad(b3, (0, D - f))
    b4r = b4.reshape(1, -1)
    fcb1r = fcb1.reshape(1, -1)
    fcb2r = fcb2.reshape(1, -1)

    t1 = pl.pallas_call(
        _prep_body,
        out_shape=jax.ShapeDtypeStruct((NP, D), jnp.float32))(xp, W1p)
    m4, _ = _gcn_sc(t1, srcm, dstm, W2f, W3f, b1p, b2p, b3p)
    z = pl.pallas_call(
        _final_body,
        out_shape=jax.ShapeDtypeStruct((NG, 10), jnp.float32))(
            m4, W4p, b4r, batch_p, fcW1, fcb1r, fcW2, fcb2r)
    return z
